# Initial kernel scaffold; baseline (speedup 1.0000x reference)
import jax
import jax.numpy as jnp
from jax.experimental import pallas as pl


def kernel(x, edge_index, W0, b0, W1, b1, W2, b2, W3, b3, W4, b4):
    raise NotImplementedError("write your pallas kernel here")


# speedup vs baseline: 14.9960x; 14.9960x over previous
import functools

import jax
import jax.numpy as jnp
from jax import lax
from jax.experimental import pallas as pl
from jax.experimental.pallas import tpu as pltpu
from jax.experimental.pallas import tpu_sc as plsc

N = 10000
E = 320000
D_IN = 128
D_HID = 32
D_OUT = 128

NPAD = 10240
NC = 2
NS = 16
NW = NC * NS
BLK = 128
NBLK = -(-E // (NW * BLK))
EPW = NBLK * BLK
EPAD = NW * EPW
ROWS = NPAD // NS

_MESH = plsc.VectorSubcoreMesh(
    core_axis_name="c", subcore_axis_name="s", num_cores=NC, num_subcores=NS
)
_SC_PARAMS = pltpu.CompilerParams(use_tc_tiling_on_sc=False)


@functools.partial(
    pl.kernel,
    out_type=jax.ShapeDtypeStruct((NC, NPAD), jnp.float32),
    mesh=_MESH,
    scratch_types=[
        pltpu.VMEM((BLK,), jnp.int32),
        pltpu.VMEM((BLK,), jnp.float32),
        pltpu.VMEM((ROWS,), jnp.float32),
        pltpu.VMEM_SHARED((NPAD,), jnp.float32),
    ],
    compiler_params=_SC_PARAMS,
)
def _deg_call(dst_hbm, deg_out, dst_v, ones_v, obuf_v, deg_sh):
    c = lax.axis_index("c")
    s = lax.axis_index("s")
    w = s * NC + c
    one = jnp.ones((16,), jnp.float32)
    for i in range(BLK // 16):
        ones_v[pl.ds(i * 16, 16)] = one

    def fill(i, carry):
        obuf_v[pl.ds(i * 16, 16)] = one
        return carry

    lax.fori_loop(0, ROWS // 16, fill, 0)
    r0 = s * ROWS
    pltpu.sync_copy(obuf_v, deg_sh.at[pl.ds(r0, ROWS)])
    plsc.subcore_barrier()

    def step(i, carry):
        base = w * EPW + i * BLK
        pltpu.sync_copy(dst_hbm.at[pl.ds(base, BLK)], dst_v)
        pltpu.sync_copy(ones_v, deg_sh.at[dst_v], add=True)
        return carry

    lax.fori_loop(0, NBLK, step, 0)
    plsc.subcore_barrier()
    pltpu.sync_copy(deg_sh.at[pl.ds(r0, ROWS)], deg_out.at[c, pl.ds(r0, ROWS)])


@functools.partial(
    pl.kernel,
    out_type=jax.ShapeDtypeStruct((NC, NPAD, D_HID), jnp.float32),
    mesh=_MESH,
    scratch_types=[
        pltpu.VMEM((BLK,), jnp.int32),
        pltpu.VMEM((BLK,), jnp.int32),
        pltpu.VMEM((BLK, D_HID), jnp.float32),
        pltpu.VMEM_SHARED((NPAD, D_HID), jnp.float32),
        pltpu.SemaphoreType.DMA,
    ],
    compiler_params=_SC_PARAMS,
)
def _prop_call(src_hbm, dst_hbm, y_hbm, z_out, src_v, dst_v, rows_v, z_sh, sem):
    c = lax.axis_index("c")
    s = lax.axis_index("s")
    w = s * NC + c
    r0 = s * ROWS
    pltpu.sync_copy(y_hbm.at[pl.ds(r0, ROWS)], z_sh.at[pl.ds(r0, ROWS)])
    plsc.subcore_barrier()

    def step(i, carry):
        base = w * EPW + i * BLK
        pltpu.sync_copy(src_hbm.at[pl.ds(base, BLK)], src_v)
        pltpu.sync_copy(dst_hbm.at[pl.ds(base, BLK)], dst_v)
        pltpu.async_copy(y_hbm.at[src_v], rows_v, sem).wait()
        pltpu.sync_copy(rows_v, z_sh.at[dst_v], add=True)
        return carry

    lax.fori_loop(0, NBLK, step, 0)
    plsc.subcore_barrier()
    pltpu.sync_copy(z_sh.at[pl.ds(r0, ROWS)], z_out.at[c, pl.ds(r0, ROWS)])


def _k0_body(x_ref, w_ref, b_ref, dega_ref, degb_ref, y_ref, dinv_ref):
    deg = dega_ref[...] + degb_ref[...] - 1.0
    dinv = lax.rsqrt(deg)
    dinv_ref[...] = dinv
    h = jnp.dot(x_ref[...], w_ref[...], preferred_element_type=jnp.float32)
    y_ref[...] = (h + b_ref[...]) * dinv


_k0 = pl.pallas_call(
    _k0_body,
    out_shape=(
        jax.ShapeDtypeStruct((NPAD, D_HID), jnp.float32),
        jax.ShapeDtypeStruct((NPAD, 1), jnp.float32),
    ),
)


def _kmid_body(za_ref, zb_ref, y_ref, dinv_ref, w_ref, b_ref, out_ref):
    dinv = dinv_ref[...]
    t = (za_ref[...] + zb_ref[...] - y_ref[...]) * dinv
    h = jnp.dot(t, w_ref[...], preferred_element_type=jnp.float32) + b_ref[...]
    out_ref[...] = jnp.maximum(h, 0.0) * dinv


_kmid = pl.pallas_call(
    _kmid_body,
    out_shape=jax.ShapeDtypeStruct((NPAD, D_HID), jnp.float32),
)


def _k3_body(za_ref, zb_ref, y_ref, dinv_ref, w3_ref, b3_ref, w4_ref, b4_ref,
             out_ref):
    t = (za_ref[...] + zb_ref[...] - y_ref[...]) * dinv_ref[...]
    h = jnp.dot(t, w3_ref[...], preferred_element_type=jnp.float32) + b3_ref[...]
    h = jnp.maximum(h, 0.0)
    out_ref[...] = (
        jnp.dot(h, w4_ref[...], preferred_element_type=jnp.float32) + b4_ref[...]
    )


_k3 = pl.pallas_call(
    _k3_body,
    out_shape=jax.ShapeDtypeStruct((NPAD, D_OUT), jnp.float32),
)


def kernel(x, edge_index, W0, b0, W1, b1, W2, b2, W3, b3, W4, b4):
    pad = jnp.full((EPAD - E,), N, dtype=jnp.int32)
    src = jnp.concatenate([edge_index[0], pad])
    dst = jnp.concatenate([edge_index[1], pad])
    xp = jnp.pad(x, ((0, NPAD - N), (0, 0)))

    deg2 = _deg_call(dst)
    dega = deg2[0][:, None]
    degb = deg2[1][:, None]
    y0, dinv = _k0(xp, W0, b0[None, :], dega, degb)
    z = _prop_call(src, dst, y0)
    y1 = _kmid(z[0], z[1], y0, dinv, W1, b1[None, :])
    z = _prop_call(src, dst, y1)
    y2 = _kmid(z[0], z[1], y1, dinv, W2, b2[None, :])
    z = _prop_call(src, dst, y2)
    out = _k3(z[0], z[1], y2, dinv, W3, b3[None, :], W4, b4[None, :])
    return out[:N]


# speedup vs baseline: 23.8466x; 1.5902x over previous
import functools

import jax
import jax.numpy as jnp
from jax import lax
from jax.experimental import pallas as pl
from jax.experimental.pallas import tpu as pltpu
from jax.experimental.pallas import tpu_sc as plsc

N = 10000
E = 320000
D_IN = 128
D_HID = 32
D_OUT = 128

NPAD = 10240
NC = 2
NS = 16
NW = NC * NS
BLK = 128
KG = 4
NGRP = 20
NBLK = KG * NGRP
EPW = NBLK * BLK
EPAD = NW * EPW
NBUF = 4
ROWS = NPAD // NS
GBYTES = BLK * D_HID * 4

_MESH = plsc.VectorSubcoreMesh(
    core_axis_name="c", subcore_axis_name="s", num_cores=NC, num_subcores=NS
)
_SC_PARAMS = pltpu.CompilerParams(use_tc_tiling_on_sc=False)


@functools.partial(
    pl.kernel,
    out_type=jax.ShapeDtypeStruct((NC, NPAD), jnp.float32),
    mesh=_MESH,
    scratch_types=[
        pltpu.VMEM((NBLK, BLK), jnp.int32),
        pltpu.VMEM((BLK,), jnp.float32),
        pltpu.VMEM((ROWS,), jnp.float32),
        pltpu.VMEM_SHARED((NPAD,), jnp.float32),
        pltpu.SemaphoreType.DMA,
    ],
    compiler_params=_SC_PARAMS,
)
def _deg_call(dst_hbm, deg_out, dst_v, ones_v, obuf_v, deg_sh, sem):
    c = lax.axis_index("c")
    s = lax.axis_index("s")
    w = s * NC + c
    one = jnp.ones((16,), jnp.float32)
    for i in range(BLK // 16):
        ones_v[pl.ds(i * 16, 16)] = one

    def fill(i, carry):
        obuf_v[pl.ds(i * 16, 16)] = one
        return carry

    lax.fori_loop(0, ROWS // 16, fill, 0)
    r0 = s * ROWS
    pltpu.sync_copy(obuf_v, deg_sh.at[pl.ds(r0, ROWS)])
    pltpu.sync_copy(dst_hbm.at[pl.ds(w * NBLK, NBLK)], dst_v)
    plsc.subcore_barrier()

    def step(i, carry):
        pltpu.async_copy(ones_v, deg_sh.at[dst_v.at[i]], sem, add=True)
        return carry

    lax.fori_loop(0, NBLK, step, 0)
    pltpu.make_async_copy(dst_hbm.at[pl.ds(w * NBLK, NBLK)], dst_v, sem).wait()
    plsc.subcore_barrier()
    pltpu.sync_copy(deg_sh.at[pl.ds(r0, ROWS)], deg_out.at[c, pl.ds(r0, ROWS)])


@functools.partial(
    pl.kernel,
    out_type=jax.ShapeDtypeStruct((NC, NPAD, D_HID), jnp.float32),
    mesh=_MESH,
    scratch_types=[
        pltpu.VMEM((NBLK, BLK), jnp.int32),
        pltpu.VMEM((NBLK, BLK), jnp.int32),
        pltpu.VMEM((NBUF, KG, BLK, D_HID), jnp.float32),
        pltpu.VMEM_SHARED((NPAD, D_HID), jnp.float32),
        pltpu.SemaphoreType.DMA((NBUF,)),
        pltpu.SemaphoreType.DMA((NBUF,)),
    ],
    compiler_params=_SC_PARAMS,
)
def _prop_call(src_hbm, dst_hbm, y_hbm, z_out, src_v, dst_v, rows_v, z_sh,
               gsem, ssem):
    c = lax.axis_index("c")
    s = lax.axis_index("s")
    w = s * NC + c
    r0 = s * ROWS
    pltpu.sync_copy(y_hbm.at[pl.ds(r0, ROWS)], z_sh.at[pl.ds(r0, ROWS)])
    pltpu.sync_copy(src_hbm.at[pl.ds(w * NBLK, NBLK)], src_v)
    pltpu.sync_copy(dst_hbm.at[pl.ds(w * NBLK, NBLK)], dst_v)
    plsc.subcore_barrier()

    def step(g, carry):
        p = lax.rem(g, NBUF)

        @pl.when(g >= NBUF)
        def _():
            for j in range(KG):
                pltpu.make_async_copy(
                    y_hbm.at[pl.ds(0, BLK)], rows_v.at[p, j], ssem.at[p]
                ).wait()

        for j in range(KG):
            pltpu.async_copy(
                y_hbm.at[src_v.at[g * KG + j]], rows_v.at[p, j], gsem.at[p]
            )

        @pl.when(g >= 1)
        def _():
            q = lax.rem(g - 1, NBUF)
            for j in range(KG):
                pltpu.make_async_copy(
                    y_hbm.at[pl.ds(0, BLK)], rows_v.at[q, j], gsem.at[q]
                ).wait()
            for j in range(KG):
                pltpu.async_copy(
                    rows_v.at[q, j], z_sh.at[dst_v.at[(g - 1) * KG + j]],
                    ssem.at[q], add=True,
                )

        return carry

    lax.fori_loop(0, NGRP, step, 0)

    q = (NGRP - 1) % NBUF
    for j in range(KG):
        pltpu.make_async_copy(
            y_hbm.at[pl.ds(0, BLK)], rows_v.at[q, j], gsem.at[q]
        ).wait()
    for j in range(KG):
        pltpu.async_copy(
            rows_v.at[q, j], z_sh.at[dst_v.at[(NGRP - 1) * KG + j]],
            ssem.at[q], add=True,
        )
    for b in range(NBUF):
        for j in range(KG):
            pltpu.make_async_copy(
                y_hbm.at[pl.ds(0, BLK)], rows_v.at[b, j], ssem.at[b]
            ).wait()
    plsc.subcore_barrier()
    pltpu.sync_copy(z_sh.at[pl.ds(r0, ROWS)], z_out.at[c, pl.ds(r0, ROWS)])


def _k0_body(x_ref, w_ref, b_ref, dega_ref, degb_ref, y_ref, dinv_ref):
    deg = dega_ref[...] + degb_ref[...] - 1.0
    dinv = lax.rsqrt(deg)
    dinv_ref[...] = dinv
    h = jnp.dot(x_ref[...], w_ref[...], preferred_element_type=jnp.float32)
    y_ref[...] = (h + b_ref[...]) * dinv


_k0 = pl.pallas_call(
    _k0_body,
    out_shape=(
        jax.ShapeDtypeStruct((NPAD, D_HID), jnp.float32),
        jax.ShapeDtypeStruct((NPAD, 1), jnp.float32),
    ),
)


def _kmid_body(za_ref, zb_ref, y_ref, dinv_ref, w_ref, b_ref, out_ref):
    dinv = dinv_ref[...]
    t = (za_ref[...] + zb_ref[...] - y_ref[...]) * dinv
    h = jnp.dot(t, w_ref[...], preferred_element_type=jnp.float32) + b_ref[...]
    out_ref[...] = jnp.maximum(h, 0.0) * dinv


_kmid = pl.pallas_call(
    _kmid_body,
    out_shape=jax.ShapeDtypeStruct((NPAD, D_HID), jnp.float32),
)


def _k3_body(za_ref, zb_ref, y_ref, dinv_ref, w3_ref, b3_ref, w4_ref, b4_ref,
             out_ref):
    t = (za_ref[...] + zb_ref[...] - y_ref[...]) * dinv_ref[...]
    h = jnp.dot(t, w3_ref[...], preferred_element_type=jnp.float32) + b3_ref[...]
    h = jnp.maximum(h, 0.0)
    out_ref[...] = (
        jnp.dot(h, w4_ref[...], preferred_element_type=jnp.float32) + b4_ref[...]
    )


_k3 = pl.pallas_call(
    _k3_body,
    out_shape=jax.ShapeDtypeStruct((NPAD, D_OUT), jnp.float32),
)


def kernel(x, edge_index, W0, b0, W1, b1, W2, b2, W3, b3, W4, b4):
    pad = jnp.full((EPAD - E,), N, dtype=jnp.int32)
    src = jnp.concatenate([edge_index[0], pad]).reshape(NW * NBLK, BLK)
    dst = jnp.concatenate([edge_index[1], pad]).reshape(NW * NBLK, BLK)
    xp = jnp.pad(x, ((0, NPAD - N), (0, 0)))

    deg2 = _deg_call(dst)
    dega = deg2[0][:, None]
    degb = deg2[1][:, None]
    y0, dinv = _k0(xp, W0, b0[None, :], dega, degb)
    z = _prop_call(src, dst, y0)
    y1 = _kmid(z[0], z[1], y0, dinv, W1, b1[None, :])
    z = _prop_call(src, dst, y1)
    y2 = _kmid(z[0], z[1], y1, dinv, W2, b2[None, :])
    z = _prop_call(src, dst, y2)
    out = _k3(z[0], z[1], y2, dinv, W3, b3[None, :], W4, b4[None, :])
    return out[:N]


# speedup vs baseline: 69.8951x; 2.9310x over previous
import functools

import jax
import jax.numpy as jnp
from jax import lax
from jax.experimental import pallas as pl
from jax.experimental.pallas import tpu as pltpu
from jax.experimental.pallas import tpu_sc as plsc

N = 10000
E = 320000
D_IN = 128
D_HID = 32
D_OUT = 128

NPAD = 10240
NC = 2
NS = 16
NW = NC * NS
BLK = 128
EBLK = E // BLK
EB_LO = EBLK // NW
EB_HI = EB_LO + 1
EXTRA = EBLK - EB_LO * NW
NBUF = 8
ROWS = NPAD // NS
NROW4 = N // 4
NPAD4 = NPAD // 4

_MESH = plsc.VectorSubcoreMesh(
    core_axis_name="c", subcore_axis_name="s", num_cores=NC, num_subcores=NS
)
_SC_PARAMS = pltpu.CompilerParams(
    use_tc_tiling_on_sc=False, needs_layout_passes=False
)


@functools.partial(
    pl.kernel,
    out_type=jax.ShapeDtypeStruct((NC, NPAD, D_HID), jnp.float32),
    mesh=_MESH,
    scratch_types=[
        pltpu.VMEM((EB_HI, BLK), jnp.int32),
        pltpu.VMEM((BLK,), jnp.float32),
        pltpu.VMEM((ROWS,), jnp.float32),
        pltpu.VMEM((ROWS, D_HID), jnp.float32),
        pltpu.VMEM_SHARED((NPAD,), jnp.float32),
        pltpu.SemaphoreType.DMA,
    ],
    compiler_params=_SC_PARAMS,
)
def _deg_call(edge_hbm, deg_out, dst_v, ones_v, obuf_v, dex_v, deg_sh, sem):
    c = lax.axis_index("c")
    s = lax.axis_index("s")
    w = s * NC + c
    nrow = jnp.where(w < EXTRA, EB_HI, EB_LO)
    base = EB_LO * w + jnp.minimum(w, EXTRA)
    one = jnp.ones((16,), jnp.float32)
    for i in range(BLK // 16):
        ones_v[pl.ds(i * 16, 16)] = one

    def fill(i, carry):
        obuf_v[pl.ds(i * 16, 16)] = one
        return carry

    lax.fori_loop(0, ROWS // 16, fill, 0)
    r0 = s * ROWS
    pltpu.sync_copy(obuf_v, deg_sh.at[pl.ds(r0, ROWS)])
    pltpu.sync_copy(edge_hbm.at[1, pl.ds(base, EB_LO)],
                    dst_v.at[pl.ds(0, EB_LO)])

    @pl.when(w < EXTRA)
    def _():
        pltpu.sync_copy(edge_hbm.at[1, pl.ds(base + EB_LO, 1)],
                        dst_v.at[pl.ds(EB_LO, 1)])

    plsc.subcore_barrier()

    def step(i, carry):
        pltpu.async_copy(ones_v, deg_sh.at[dst_v.at[i]], sem, add=True)
        return carry

    lax.fori_loop(0, nrow, step, 0)

    def drain(i, carry):
        pltpu.make_async_copy(edge_hbm.at[1, pl.ds(0, 1)],
                              dst_v.at[pl.ds(0, 1)], sem).wait()
        return carry

    lax.fori_loop(0, nrow, drain, 0)
    plsc.subcore_barrier()
    pltpu.sync_copy(deg_sh.at[pl.ds(r0, ROWS)], obuf_v)

    def expand(i, carry):
        v = plsc.load_gather(obuf_v, [jnp.full((16,), i, jnp.int32)])
        dex_v[i, pl.ds(0, 16)] = v
        dex_v[i, pl.ds(16, 16)] = v
        return carry

    lax.fori_loop(0, ROWS, expand, 0)
    pltpu.sync_copy(dex_v, deg_out.at[c, pl.ds(r0, ROWS)])


@functools.partial(
    pl.kernel,
    out_type=jax.ShapeDtypeStruct((NC, NPAD, D_HID), jnp.float32),
    mesh=_MESH,
    scratch_types=[
        pltpu.VMEM((EB_HI, BLK), jnp.int32),
        pltpu.VMEM((EB_HI, BLK), jnp.int32),
        pltpu.VMEM((NBUF, BLK, D_HID), jnp.float32),
        pltpu.VMEM_SHARED((NPAD, D_HID), jnp.float32),
        pltpu.VMEM_SHARED((NPAD, D_HID), jnp.float32),
        pltpu.SemaphoreType.DMA((NBUF,)),
        pltpu.SemaphoreType.DMA((NBUF,)),
    ],
    compiler_params=_SC_PARAMS,
)
def _prop_call(edge_hbm, y_hbm, z_out, src_v, dst_v, rows_v, z_sh, y_sh,
               gsem, ssem):
    c = lax.axis_index("c")
    s = lax.axis_index("s")
    w = s * NC + c
    nrow = jnp.where(w < EXTRA, EB_HI, EB_LO)
    base = EB_LO * w + jnp.minimum(w, EXTRA)
    r0 = s * ROWS
    pltpu.sync_copy(y_hbm.at[pl.ds(r0, ROWS)], y_sh.at[pl.ds(r0, ROWS)])
    pltpu.sync_copy(y_hbm.at[pl.ds(r0, ROWS)], z_sh.at[pl.ds(r0, ROWS)])
    pltpu.sync_copy(edge_hbm.at[0, pl.ds(base, EB_LO)],
                    src_v.at[pl.ds(0, EB_LO)])
    pltpu.sync_copy(edge_hbm.at[1, pl.ds(base, EB_LO)],
                    dst_v.at[pl.ds(0, EB_LO)])

    @pl.when(w < EXTRA)
    def _():
        pltpu.sync_copy(edge_hbm.at[0, pl.ds(base + EB_LO, 1)],
                        src_v.at[pl.ds(EB_LO, 1)])
        pltpu.sync_copy(edge_hbm.at[1, pl.ds(base + EB_LO, 1)],
                        dst_v.at[pl.ds(EB_LO, 1)])

    plsc.subcore_barrier()

    def step(g, carry):
        p = lax.rem(g, NBUF)

        @pl.when(g >= NBUF)
        def _():
            pltpu.make_async_copy(
                y_hbm.at[pl.ds(0, BLK)], rows_v.at[p], ssem.at[p]
            ).wait()

        pltpu.async_copy(y_sh.at[src_v.at[g]], rows_v.at[p], gsem.at[p])

        @pl.when(g >= 1)
        def _():
            q = lax.rem(g - 1, NBUF)
            pltpu.make_async_copy(
                y_hbm.at[pl.ds(0, BLK)], rows_v.at[q], gsem.at[q]
            ).wait()
            pltpu.async_copy(
                rows_v.at[q], z_sh.at[dst_v.at[g - 1]], ssem.at[q], add=True
            )

        return carry

    lax.fori_loop(0, nrow, step, 0)

    q = lax.rem(nrow - 1, NBUF)
    pltpu.make_async_copy(y_hbm.at[pl.ds(0, BLK)], rows_v.at[q],
                          gsem.at[q]).wait()
    pltpu.async_copy(rows_v.at[q], z_sh.at[dst_v.at[nrow - 1]], ssem.at[q],
                     add=True)
    for b in range(NBUF):
        pltpu.make_async_copy(
            y_hbm.at[pl.ds(0, BLK)], rows_v.at[b], ssem.at[b]
        ).wait()
    plsc.subcore_barrier()
    pltpu.sync_copy(z_sh.at[pl.ds(r0, ROWS)], z_out.at[c, pl.ds(r0, ROWS)])


def _k0_body(x4_ref, w_ref, b_ref, deg_ref, y_ref, dinv_ref):
    dinv = lax.rsqrt(deg_ref[0] + deg_ref[1] - 1.0)
    dinv_ref[...] = dinv
    h = jnp.dot(x4_ref[...], w_ref[...], preferred_element_type=jnp.float32)
    y_ref[0:NROW4, :] = (h + b_ref[...]) * dinv[0:NROW4, :]
    y_ref[NROW4:NPAD4, :] = jnp.zeros((NPAD4 - NROW4, D_IN), jnp.float32)


_k0 = pl.pallas_call(
    _k0_body,
    out_shape=(
        jax.ShapeDtypeStruct((NPAD4, D_IN), jnp.float32),
        jax.ShapeDtypeStruct((NPAD4, D_IN), jnp.float32),
    ),
)


def _kmid_body(z_ref, y_ref, dinv_ref, w_ref, b_ref, out_ref):
    dinv = dinv_ref[...]
    t = (z_ref[0] + z_ref[1] - y_ref[...]) * dinv
    h = jnp.dot(t, w_ref[...], preferred_element_type=jnp.float32) + b_ref[...]
    out_ref[...] = jnp.maximum(h, 0.0) * dinv


_kmid = pl.pallas_call(
    _kmid_body,
    out_shape=jax.ShapeDtypeStruct((NPAD4, D_IN), jnp.float32),
)


def _k3_body(z_ref, y_ref, dinv_ref, w3_ref, b3_ref, w4_ref, b4_ref, out_ref):
    t = (z_ref[0] + z_ref[1] - y_ref[...]) * dinv_ref[...]
    h = jnp.dot(t, w3_ref[...], preferred_element_type=jnp.float32) + b3_ref[...]
    h = jnp.maximum(h, 0.0)
    out_ref[...] = (
        jnp.dot(h[0:NROW4, :], w4_ref[...], preferred_element_type=jnp.float32)
        + b4_ref[...]
    )


_k3 = pl.pallas_call(
    _k3_body,
    out_shape=jax.ShapeDtypeStruct((NROW4, 4 * D_OUT), jnp.float32),
)


def kernel(x, edge_index, W0, b0, W1, b1, W2, b2, W3, b3, W4, b4):
    edge_r = edge_index.reshape(2, EBLK, BLK)
    x4 = x.reshape(NROW4, 4 * D_IN)
    eye4 = jnp.eye(4, dtype=jnp.float32)
    w0b = jnp.kron(eye4, W0)
    w1b = jnp.kron(eye4, W1)
    w2b = jnp.kron(eye4, W2)
    w3b = jnp.kron(eye4, W3)
    w4b = jnp.kron(eye4, W4)
    b0t = jnp.tile(b0, 4)[None, :]
    b1t = jnp.tile(b1, 4)[None, :]
    b2t = jnp.tile(b2, 4)[None, :]
    b3t = jnp.tile(b3, 4)[None, :]
    b4t = jnp.tile(b4, 4)[None, :]

    deg = _deg_call(edge_r).reshape(NC, NPAD4, D_IN)
    y0, dinv = _k0(x4, w0b, b0t, deg)
    z = _prop_call(edge_r, y0.reshape(NPAD, D_HID)).reshape(NC, NPAD4, D_IN)
    y1 = _kmid(z, y0, dinv, w1b, b1t)
    z = _prop_call(edge_r, y1.reshape(NPAD, D_HID)).reshape(NC, NPAD4, D_IN)
    y2 = _kmid(z, y1, dinv, w2b, b2t)
    z = _prop_call(edge_r, y2.reshape(NPAD, D_HID)).reshape(NC, NPAD4, D_IN)
    out4 = _k3(z, y2, dinv, w3b, b3t, w4b, b4t)
    return out4.reshape(N, D_OUT)


# speedup vs baseline: 74.3548x; 1.0638x over previous
import functools

import jax
import jax.numpy as jnp
from jax import lax
from jax.experimental import pallas as pl
from jax.experimental.pallas import tpu as pltpu
from jax.experimental.pallas import tpu_sc as plsc

N = 10000
E = 320000
D_IN = 128
D_HID = 32
D_OUT = 128

NPAD = 10240
NC = 2
NS = 16
NW = NC * NS
BLK = 128
EBLK = E // BLK
EB_LO = EBLK // NW
EB_HI = EB_LO + 1
EXTRA = EBLK - EB_LO * NW
NBUF = 8
LAG = 3
ROWS = NPAD // NS
NROW4 = N // 4
NPAD4 = NPAD // 4

_MESH = plsc.VectorSubcoreMesh(
    core_axis_name="c", subcore_axis_name="s", num_cores=NC, num_subcores=NS
)
_SC_PARAMS = pltpu.CompilerParams(
    use_tc_tiling_on_sc=False, needs_layout_passes=False
)


@functools.partial(
    pl.kernel,
    out_type=jax.ShapeDtypeStruct((NC, NPAD, D_HID), jnp.float32),
    mesh=_MESH,
    scratch_types=[
        pltpu.VMEM((EB_HI, BLK), jnp.int32),
        pltpu.VMEM((BLK,), jnp.float32),
        pltpu.VMEM((ROWS,), jnp.float32),
        pltpu.VMEM((ROWS, D_HID), jnp.float32),
        pltpu.VMEM_SHARED((NPAD,), jnp.float32),
        pltpu.SemaphoreType.DMA,
    ],
    compiler_params=_SC_PARAMS,
)
def _deg_call(edge_hbm, deg_out, dst_v, ones_v, obuf_v, dex_v, deg_sh, sem):
    c = lax.axis_index("c")
    s = lax.axis_index("s")
    w = s * NC + c
    nrow = jnp.where(w < EXTRA, EB_HI, EB_LO)
    base = EB_LO * w + jnp.minimum(w, EXTRA)
    one = jnp.ones((16,), jnp.float32)
    for i in range(BLK // 16):
        ones_v[pl.ds(i * 16, 16)] = one

    def fill(i, carry):
        obuf_v[pl.ds(i * 16, 16)] = one
        return carry

    lax.fori_loop(0, ROWS // 16, fill, 0)
    r0 = s * ROWS
    pltpu.sync_copy(obuf_v, deg_sh.at[pl.ds(r0, ROWS)])
    pltpu.sync_copy(edge_hbm.at[1, pl.ds(base, EB_LO)],
                    dst_v.at[pl.ds(0, EB_LO)])

    @pl.when(w < EXTRA)
    def _():
        pltpu.sync_copy(edge_hbm.at[1, pl.ds(base + EB_LO, 1)],
                        dst_v.at[pl.ds(EB_LO, 1)])

    plsc.subcore_barrier()

    def step(i, carry):
        pltpu.async_copy(ones_v, deg_sh.at[dst_v.at[i]], sem, add=True)
        return carry

    lax.fori_loop(0, nrow, step, 0)

    def drain(i, carry):
        pltpu.make_async_copy(edge_hbm.at[1, pl.ds(0, 1)],
                              dst_v.at[pl.ds(0, 1)], sem).wait()
        return carry

    lax.fori_loop(0, nrow, drain, 0)
    plsc.subcore_barrier()
    pltpu.sync_copy(deg_sh.at[pl.ds(r0, ROWS)], obuf_v)

    def expand(i, carry):
        v = plsc.load_gather(obuf_v, [jnp.full((16,), i, jnp.int32)])
        dex_v[i, pl.ds(0, 16)] = v
        dex_v[i, pl.ds(16, 16)] = v
        return carry

    lax.fori_loop(0, ROWS, expand, 0)
    pltpu.sync_copy(dex_v, deg_out.at[c, pl.ds(r0, ROWS)])


@functools.partial(
    pl.kernel,
    out_type=jax.ShapeDtypeStruct((NC, NPAD, D_HID), jnp.float32),
    mesh=_MESH,
    scratch_types=[
        pltpu.VMEM((EB_HI, BLK), jnp.int32),
        pltpu.VMEM((EB_HI, BLK), jnp.int32),
        pltpu.VMEM((NBUF, BLK, D_HID), jnp.float32),
        pltpu.VMEM_SHARED((NPAD, D_HID), jnp.float32),
        pltpu.VMEM_SHARED((NPAD, D_HID), jnp.float32),
        pltpu.SemaphoreType.DMA((NBUF,)),
        pltpu.SemaphoreType.DMA((NBUF,)),
    ],
    compiler_params=_SC_PARAMS,
)
def _prop_call(edge_hbm, y_hbm, z_out, src_v, dst_v, rows_v, z_sh, y_sh,
               gsem, ssem):
    c = lax.axis_index("c")
    s = lax.axis_index("s")
    w = s * NC + c
    nrow = jnp.where(w < EXTRA, EB_HI, EB_LO)
    base = EB_LO * w + jnp.minimum(w, EXTRA)
    r0 = s * ROWS
    d1 = pltpu.async_copy(y_hbm.at[pl.ds(r0, ROWS)], y_sh.at[pl.ds(r0, ROWS)],
                          gsem.at[0])
    d2 = pltpu.async_copy(y_hbm.at[pl.ds(r0, ROWS)], z_sh.at[pl.ds(r0, ROWS)],
                          gsem.at[1])
    d3 = pltpu.async_copy(edge_hbm.at[0, pl.ds(base, EB_LO)],
                          src_v.at[pl.ds(0, EB_LO)], gsem.at[2])
    d4 = pltpu.async_copy(edge_hbm.at[1, pl.ds(base, EB_LO)],
                          dst_v.at[pl.ds(0, EB_LO)], gsem.at[3])

    @pl.when(w < EXTRA)
    def _():
        pltpu.sync_copy(edge_hbm.at[0, pl.ds(base + EB_LO, 1)],
                        src_v.at[pl.ds(EB_LO, 1)])
        pltpu.sync_copy(edge_hbm.at[1, pl.ds(base + EB_LO, 1)],
                        dst_v.at[pl.ds(EB_LO, 1)])

    d1.wait()
    d2.wait()
    d3.wait()
    d4.wait()
    plsc.subcore_barrier()

    def gather(g):
        p = lax.rem(g, NBUF)
        pltpu.async_copy(y_sh.at[src_v.at[g]], rows_v.at[p], gsem.at[p])

    def scatter(g):
        q = lax.rem(g, NBUF)
        pltpu.make_async_copy(
            y_hbm.at[pl.ds(0, BLK)], rows_v.at[q], gsem.at[q]
        ).wait()
        pltpu.async_copy(
            rows_v.at[q], z_sh.at[dst_v.at[g]], ssem.at[q], add=True
        )

    def step(g, carry):
        @pl.when(g >= NBUF)
        def _():
            p = lax.rem(g, NBUF)
            pltpu.make_async_copy(
                y_hbm.at[pl.ds(0, BLK)], rows_v.at[p], ssem.at[p]
            ).wait()

        gather(g)

        @pl.when(g >= LAG)
        def _():
            scatter(g - LAG)

        return carry

    lax.fori_loop(0, nrow, step, 0)

    def tail(g, carry):
        scatter(g)
        return carry

    lax.fori_loop(nrow - LAG, nrow, tail, 0)
    for b in range(NBUF):
        pltpu.make_async_copy(
            y_hbm.at[pl.ds(0, BLK)], rows_v.at[b], ssem.at[b]
        ).wait()
    plsc.subcore_barrier()
    pltpu.sync_copy(z_sh.at[pl.ds(r0, ROWS)], z_out.at[c, pl.ds(r0, ROWS)])


def _k0_body(x4_ref, w_ref, b_ref, deg_ref, y_ref, dinv_ref):
    dinv = lax.rsqrt(deg_ref[0] + deg_ref[1] - 1.0)
    dinv_ref[...] = dinv
    h = jnp.dot(x4_ref[...], w_ref[...], preferred_element_type=jnp.float32)
    y_ref[0:NROW4, :] = (h + b_ref[...]) * dinv[0:NROW4, :]
    y_ref[NROW4:NPAD4, :] = jnp.zeros((NPAD4 - NROW4, D_IN), jnp.float32)


_k0 = pl.pallas_call(
    _k0_body,
    out_shape=(
        jax.ShapeDtypeStruct((NPAD4, D_IN), jnp.float32),
        jax.ShapeDtypeStruct((NPAD4, D_IN), jnp.float32),
    ),
)


def _kmid_body(z_ref, y_ref, dinv_ref, w_ref, b_ref, out_ref):
    dinv = dinv_ref[...]
    t = (z_ref[0] + z_ref[1] - y_ref[...]) * dinv
    h = jnp.dot(t, w_ref[...], preferred_element_type=jnp.float32) + b_ref[...]
    out_ref[...] = jnp.maximum(h, 0.0) * dinv


_kmid = pl.pallas_call(
    _kmid_body,
    out_shape=jax.ShapeDtypeStruct((NPAD4, D_IN), jnp.float32),
)


def _k3_body(z_ref, y_ref, dinv_ref, w3_ref, b3_ref, w4_ref, b4_ref, out_ref):
    t = (z_ref[0] + z_ref[1] - y_ref[...]) * dinv_ref[...]
    h = jnp.dot(t, w3_ref[...], preferred_element_type=jnp.float32) + b3_ref[...]
    h = jnp.maximum(h, 0.0)
    out_ref[...] = (
        jnp.dot(h[0:NROW4, :], w4_ref[...], preferred_element_type=jnp.float32)
        + b4_ref[...]
    )


_k3 = pl.pallas_call(
    _k3_body,
    out_shape=jax.ShapeDtypeStruct((NROW4, 4 * D_OUT), jnp.float32),
)


def kernel(x, edge_index, W0, b0, W1, b1, W2, b2, W3, b3, W4, b4):
    edge_r = edge_index.reshape(2, EBLK, BLK)
    x4 = x.reshape(NROW4, 4 * D_IN)
    eye4 = jnp.eye(4, dtype=jnp.float32)
    w0b = jnp.kron(eye4, W0)
    w1b = jnp.kron(eye4, W1)
    w2b = jnp.kron(eye4, W2)
    w3b = jnp.kron(eye4, W3)
    w4b = jnp.kron(eye4, W4)
    b0t = jnp.tile(b0, 4)[None, :]
    b1t = jnp.tile(b1, 4)[None, :]
    b2t = jnp.tile(b2, 4)[None, :]
    b3t = jnp.tile(b3, 4)[None, :]
    b4t = jnp.tile(b4, 4)[None, :]

    deg = _deg_call(edge_r).reshape(NC, NPAD4, D_IN)
    y0, dinv = _k0(x4, w0b, b0t, deg)
    z = _prop_call(edge_r, y0.reshape(NPAD, D_HID)).reshape(NC, NPAD4, D_IN)
    y1 = _kmid(z, y0, dinv, w1b, b1t)
    z = _prop_call(edge_r, y1.reshape(NPAD, D_HID)).reshape(NC, NPAD4, D_IN)
    y2 = _kmid(z, y1, dinv, w2b, b2t)
    z = _prop_call(edge_r, y2.reshape(NPAD, D_HID)).reshape(NC, NPAD4, D_IN)
    out4 = _k3(z, y2, dinv, w3b, b3t, w4b, b4t)
    return out4.reshape(N, D_OUT)


# speedup vs baseline: 74.3970x; 1.0006x over previous
import functools

import jax
import jax.numpy as jnp
from jax import lax
from jax.experimental import pallas as pl
from jax.experimental.pallas import tpu as pltpu
from jax.experimental.pallas import tpu_sc as plsc

N = 10000
E = 320000
D_IN = 128
D_HID = 32
D_OUT = 128

NPAD = 10240
NC = 2
NS = 16
NW = NC * NS
BLK = 128
EBLK = E // BLK
EB_LO = EBLK // NW
EB_HI = EB_LO + 1
EXTRA = EBLK - EB_LO * NW
NBUF = 8
LAG = 3
ROWS = NPAD // NS
NROW4 = N // 4
NPAD4 = NPAD // 4

_MESH = plsc.VectorSubcoreMesh(
    core_axis_name="c", subcore_axis_name="s", num_cores=NC, num_subcores=NS
)
_SC_PARAMS = pltpu.CompilerParams(
    use_tc_tiling_on_sc=False, needs_layout_passes=False
)


@functools.partial(
    pl.kernel,
    out_type=jax.ShapeDtypeStruct((NC, NPAD, D_HID), jnp.float32),
    mesh=_MESH,
    scratch_types=[
        pltpu.VMEM((EB_HI, BLK), jnp.int32),
        pltpu.VMEM((BLK,), jnp.float32),
        pltpu.VMEM((ROWS,), jnp.float32),
        pltpu.VMEM((ROWS, D_HID), jnp.float32),
        pltpu.VMEM_SHARED((NPAD,), jnp.float32),
        pltpu.SemaphoreType.DMA,
    ],
    compiler_params=_SC_PARAMS,
)
def _deg_call(edge_hbm, deg_out, dst_v, ones_v, obuf_v, dex_v, deg_sh, sem):
    c = lax.axis_index("c")
    s = lax.axis_index("s")
    w = s * NC + c
    nrow = jnp.where(w < EXTRA, EB_HI, EB_LO)
    base = EB_LO * w + jnp.minimum(w, EXTRA)
    one = jnp.ones((16,), jnp.float32)
    for i in range(BLK // 16):
        ones_v[pl.ds(i * 16, 16)] = one

    def fill(i, carry):
        obuf_v[pl.ds(i * 16, 16)] = one
        return carry

    lax.fori_loop(0, ROWS // 16, fill, 0)
    r0 = s * ROWS
    pltpu.sync_copy(obuf_v, deg_sh.at[pl.ds(r0, ROWS)])
    pltpu.sync_copy(edge_hbm.at[1, pl.ds(base, EB_LO)],
                    dst_v.at[pl.ds(0, EB_LO)])

    @pl.when(w < EXTRA)
    def _():
        pltpu.sync_copy(edge_hbm.at[1, pl.ds(base + EB_LO, 1)],
                        dst_v.at[pl.ds(EB_LO, 1)])

    plsc.subcore_barrier()

    def step(i, carry):
        pltpu.async_copy(ones_v, deg_sh.at[dst_v.at[i]], sem, add=True)
        return carry

    lax.fori_loop(0, nrow, step, 0)

    def drain(i, carry):
        pltpu.make_async_copy(edge_hbm.at[1, pl.ds(0, 1)],
                              dst_v.at[pl.ds(0, 1)], sem).wait()
        return carry

    lax.fori_loop(0, nrow, drain, 0)
    plsc.subcore_barrier()
    pltpu.sync_copy(deg_sh.at[pl.ds(r0, ROWS)], obuf_v)

    def expand(i, carry):
        v = plsc.load_gather(obuf_v, [jnp.full((16,), i, jnp.int32)])
        dex_v[i, pl.ds(0, 16)] = v
        dex_v[i, pl.ds(16, 16)] = v
        return carry

    lax.fori_loop(0, ROWS, expand, 0)
    pltpu.sync_copy(dex_v, deg_out.at[c, pl.ds(r0, ROWS)])


@functools.partial(
    pl.kernel,
    out_type=jax.ShapeDtypeStruct((NC, NPAD, D_HID), jnp.float32),
    mesh=_MESH,
    scratch_types=[
        pltpu.VMEM((EB_HI, BLK), jnp.int32),
        pltpu.VMEM((EB_HI, BLK), jnp.int32),
        pltpu.VMEM((NBUF, BLK, D_HID), jnp.float32),
        pltpu.VMEM_SHARED((NPAD, D_HID), jnp.float32),
        pltpu.VMEM_SHARED((NPAD, D_HID), jnp.float32),
        pltpu.SemaphoreType.DMA((NBUF,)),
        pltpu.SemaphoreType.DMA((NBUF,)),
    ],
    compiler_params=_SC_PARAMS,
)
def _prop_call(edge_hbm, y_hbm, z_out, src_v, dst_v, rows_v, z_sh, y_sh,
               gsem, ssem):
    c = lax.axis_index("c")
    s = lax.axis_index("s")
    w = s * NC + c
    nrow = jnp.where(w < EXTRA, EB_HI, EB_LO)
    base = EB_LO * w + jnp.minimum(w, EXTRA)
    r0 = s * ROWS
    d1 = pltpu.async_copy(y_hbm.at[pl.ds(r0, ROWS)], y_sh.at[pl.ds(r0, ROWS)],
                          gsem.at[0])
    d2 = pltpu.async_copy(y_hbm.at[pl.ds(r0, ROWS)], z_sh.at[pl.ds(r0, ROWS)],
                          gsem.at[1])
    d3 = pltpu.async_copy(edge_hbm.at[0, pl.ds(base, EB_LO)],
                          src_v.at[pl.ds(0, EB_LO)], gsem.at[2])
    d4 = pltpu.async_copy(edge_hbm.at[1, pl.ds(base, EB_LO)],
                          dst_v.at[pl.ds(0, EB_LO)], gsem.at[3])

    @pl.when(w < EXTRA)
    def _():
        pltpu.sync_copy(edge_hbm.at[0, pl.ds(base + EB_LO, 1)],
                        src_v.at[pl.ds(EB_LO, 1)])
        pltpu.sync_copy(edge_hbm.at[1, pl.ds(base + EB_LO, 1)],
                        dst_v.at[pl.ds(EB_LO, 1)])

    d1.wait()
    d2.wait()
    d3.wait()
    d4.wait()
    plsc.subcore_barrier()

    def gather(g):
        p = lax.rem(g, NBUF)
        pltpu.async_copy(y_sh.at[src_v.at[g]], rows_v.at[p], gsem.at[p])

    def scatter(g):
        q = lax.rem(g, NBUF)
        pltpu.make_async_copy(
            y_hbm.at[pl.ds(0, BLK)], rows_v.at[q], gsem.at[q]
        ).wait()
        pltpu.async_copy(
            rows_v.at[q], z_sh.at[dst_v.at[g]], ssem.at[q], add=True
        )

    def step(g, carry):
        @pl.when(g >= NBUF)
        def _():
            p = lax.rem(g, NBUF)
            pltpu.make_async_copy(
                y_hbm.at[pl.ds(0, BLK)], rows_v.at[p], ssem.at[p]
            ).wait()

        gather(g)

        @pl.when(g >= LAG)
        def _():
            scatter(g - LAG)

        return carry

    lax.fori_loop(0, nrow, step, 0)

    def tail(g, carry):
        scatter(g)
        return carry

    lax.fori_loop(nrow - LAG, nrow, tail, 0)
    for b in range(NBUF):
        pltpu.make_async_copy(
            y_hbm.at[pl.ds(0, BLK)], rows_v.at[b], ssem.at[b]
        ).wait()
    plsc.subcore_barrier()
    pltpu.sync_copy(z_sh.at[pl.ds(r0, ROWS)], z_out.at[c, pl.ds(r0, ROWS)])


def _k0_body(x4_ref, w_ref, b_ref, deg_ref, y_ref, dinv_ref):
    dinv = lax.rsqrt(deg_ref[0] + deg_ref[1] - 1.0)
    dinv_ref[...] = dinv
    h = jnp.dot(x4_ref[...], w_ref[...], preferred_element_type=jnp.float32)
    y_ref[0:NROW4, :] = (h + b_ref[...]) * dinv[0:NROW4, :]
    y_ref[NROW4:NPAD4, :] = jnp.zeros((NPAD4 - NROW4, D_IN), jnp.float32)


_k0 = pl.pallas_call(
    _k0_body,
    out_shape=(
        jax.ShapeDtypeStruct((NPAD4, D_IN), jnp.float32),
        jax.ShapeDtypeStruct((NPAD4, D_IN), jnp.float32),
    ),
)


def _kmid_body(z_ref, y_ref, dinv_ref, w_ref, b_ref, out_ref):
    dinv = dinv_ref[...]
    t = (z_ref[0] + z_ref[1] - y_ref[...]) * dinv
    h = jnp.dot(t, w_ref[...], preferred_element_type=jnp.float32) + b_ref[...]
    out_ref[...] = jnp.maximum(h, 0.0) * dinv


_kmid = pl.pallas_call(
    _kmid_body,
    out_shape=jax.ShapeDtypeStruct((NPAD4, D_IN), jnp.float32),
)


def _k3_body(z_ref, y_ref, dinv_ref, w3_ref, b3_ref, w4_ref, b4_ref, out_ref):
    t = (z_ref[0] + z_ref[1] - y_ref[...]) * dinv_ref[...]
    h = jnp.dot(t, w3_ref[...], preferred_element_type=jnp.float32) + b3_ref[...]
    h = jnp.maximum(h, 0.0)
    out_ref[...] = (
        jnp.dot(h[0:NROW4, :], w4_ref[...], preferred_element_type=jnp.float32)
        + b4_ref[...]
    )


_k3 = pl.pallas_call(
    _k3_body,
    out_shape=jax.ShapeDtypeStruct((NROW4, 4 * D_OUT), jnp.float32),
)


def kernel(x, edge_index, W0, b0, W1, b1, W2, b2, W3, b3, W4, b4):
    edge_r = edge_index.reshape(2, EBLK, BLK)
    deg = _deg_call(edge_r).reshape(NC, NPAD4, D_IN)
    x4 = x.reshape(NROW4, 4 * D_IN)
    eye4 = jnp.eye(4, dtype=jnp.float32)
    w0b = jnp.kron(eye4, W0)
    w1b = jnp.kron(eye4, W1)
    w2b = jnp.kron(eye4, W2)
    w3b = jnp.kron(eye4, W3)
    w4b = jnp.kron(eye4, W4)
    b0t = jnp.tile(b0, 4)[None, :]
    b1t = jnp.tile(b1, 4)[None, :]
    b2t = jnp.tile(b2, 4)[None, :]
    b3t = jnp.tile(b3, 4)[None, :]
    b4t = jnp.tile(b4, 4)[None, :]

    y0, dinv = _k0(x4, w0b, b0t, deg)
    z = _prop_call(edge_r, y0.reshape(NPAD, D_HID)).reshape(NC, NPAD4, D_IN)
    y1 = _kmid(z, y0, dinv, w1b, b1t)
    z = _prop_call(edge_r, y1.reshape(NPAD, D_HID)).reshape(NC, NPAD4, D_IN)
    y2 = _kmid(z, y1, dinv, w2b, b2t)
    z = _prop_call(edge_r, y2.reshape(NPAD, D_HID)).reshape(NC, NPAD4, D_IN)
    out4 = _k3(z, y2, dinv, w3b, b3t, w4b, b4t)
    return out4.reshape(N, D_OUT)
